# Initial kernel scaffold; baseline (speedup 1.0000x reference)
import jax
import jax.numpy as jnp
from jax.experimental import pallas as pl


def kernel(pred_cls, pred_reg, anchors):
    raise NotImplementedError("write your pallas kernel here")


# speedup vs baseline: 18.8443x; 18.8443x over previous
import functools

import jax
import jax.numpy as jnp
from jax.experimental import pallas as pl

B = 32
N = 3125
NP = 3200
PRE_K = 2000
POST_K = 300
ROWS_PAD = 304
THRESH = 0.7
SZM1 = 254.0
NEG = -1.0e30


def _f32_sort_key(s):
    u = jax.lax.bitcast_convert_type(s, jnp.int32)
    flipped = jnp.bitwise_xor(jnp.invert(u), jnp.int32(-2147483648))
    return jnp.where(u >= 0, u, flipped)


def _nms_body(cls0_ref, cls1_ref, d0_ref, d1_ref, d2_ref, d3_ref, anc_ref,
              os_ref, ox1_ref, oy1_ref, ox2_ref, oy2_ref):
    c0 = cls0_ref[:, :]
    c1 = cls1_ref[:, :]
    m = jnp.maximum(c0, c1)
    s = (c1 - m) - jnp.log(jnp.exp(c0 - m) + jnp.exp(c1 - m))
    col = jax.lax.broadcasted_iota(jnp.int32, (B, NP), 1)
    s = jnp.where(col < N, s, NEG)

    aw = anc_ref[2:3, :] - anc_ref[0:1, :] + 1.0
    ah = anc_ref[3:4, :] - anc_ref[1:2, :] + 1.0
    acx = anc_ref[0:1, :] + 0.5 * aw
    acy = anc_ref[1:2, :] + 0.5 * ah
    pcx = d0_ref[:, :] * aw + acx
    pcy = d1_ref[:, :] * ah + acy
    pw = jnp.exp(d2_ref[:, :]) * aw
    ph = jnp.exp(d3_ref[:, :]) * ah
    x1 = jnp.clip(pcx - 0.5 * pw, 0.0, SZM1)
    y1 = jnp.clip(pcy - 0.5 * ph, 0.0, SZM1)
    x2 = jnp.clip(pcx + 0.5 * pw, 0.0, SZM1)
    y2 = jnp.clip(pcy + 0.5 * ph, 0.0, SZM1)
    area = (x2 - x1 + 1.0) * (y2 - y1 + 1.0)

    key = _f32_sort_key(s)
    lo0 = jnp.min(key, axis=1, keepdims=True)
    hi0 = jnp.max(key, axis=1, keepdims=True) + 1

    def bs_step(_, lh):
        lo, hi = lh
        mid = lo + ((hi - lo) >> 1)
        cnt = jnp.sum((key >= mid).astype(jnp.int32), axis=1, keepdims=True)
        ge = cnt >= PRE_K
        return jnp.where(ge, mid, lo), jnp.where(ge, hi, mid)

    lo, _ = jax.lax.fori_loop(0, 32, bs_step, (lo0, hi0))
    valid0 = (key >= lo).astype(jnp.int32)

    def nms_step(i, valid):
        masked = jnp.where(valid > 0, s, NEG)
        mx = jnp.max(masked, axis=1, keepdims=True)
        has = mx > -1.0e29
        sel = jnp.min(jnp.where(masked == mx, col, NP), axis=1, keepdims=True)
        onehot = col == sel
        bx1 = jnp.sum(jnp.where(onehot, x1, 0.0), axis=1, keepdims=True)
        by1 = jnp.sum(jnp.where(onehot, y1, 0.0), axis=1, keepdims=True)
        bx2 = jnp.sum(jnp.where(onehot, x2, 0.0), axis=1, keepdims=True)
        by2 = jnp.sum(jnp.where(onehot, y2, 0.0), axis=1, keepdims=True)
        barea = (bx2 - bx1 + 1.0) * (by2 - by1 + 1.0)
        xx1 = jnp.maximum(bx1, x1)
        yy1 = jnp.maximum(by1, y1)
        xx2 = jnp.minimum(bx2, x2)
        yy2 = jnp.minimum(by2, y2)
        iw = jnp.maximum(xx2 - xx1 + 1.0, 0.0)
        ih = jnp.maximum(yy2 - yy1 + 1.0, 0.0)
        inter = iw * ih
        keep_ok = (inter * (1.0 + THRESH) <= THRESH * (barea + area)).astype(jnp.int32)
        new_valid = jnp.where(has, valid * keep_ok, valid)
        z = jnp.zeros_like(mx)
        os_ref[pl.ds(i, 1), :] = jnp.where(has, mx, z).reshape(1, B)
        ox1_ref[pl.ds(i, 1), :] = jnp.where(has, bx1, z).reshape(1, B)
        oy1_ref[pl.ds(i, 1), :] = jnp.where(has, by1, z).reshape(1, B)
        ox2_ref[pl.ds(i, 1), :] = jnp.where(has, bx2, z).reshape(1, B)
        oy2_ref[pl.ds(i, 1), :] = jnp.where(has, by2, z).reshape(1, B)
        return new_valid

    jax.lax.fori_loop(0, ROWS_PAD, nms_step, valid0)


@jax.jit
def kernel(pred_cls, pred_reg, anchors):
    pc = pred_cls.reshape(B, 2, 5, 625)
    pr = pred_reg.reshape(B, 4, 5, 625)
    pad = ((0, 0), (0, NP - N))
    cls0 = jnp.pad(pc[:, 0].reshape(B, N), pad)
    cls1 = jnp.pad(pc[:, 1].reshape(B, N), pad)
    d0 = jnp.pad(pr[:, 0].reshape(B, N), pad)
    d1 = jnp.pad(pr[:, 1].reshape(B, N), pad)
    d2 = jnp.pad(pr[:, 2].reshape(B, N), pad)
    d3 = jnp.pad(pr[:, 3].reshape(B, N), pad)
    anc = jnp.pad(anchors.T, ((0, 0), (0, NP - N)))

    out_sh = [jax.ShapeDtypeStruct((ROWS_PAD, B), jnp.float32)] * 5
    outs = pl.pallas_call(_nms_body, out_shape=out_sh)(
        cls0, cls1, d0, d1, d2, d3, anc)
    rows = jnp.stack([o.T[:, :POST_K] for o in outs], axis=-1)
    return rows


# speedup vs baseline: 41.5365x; 2.2042x over previous
import functools

import jax
import jax.numpy as jnp
from jax import lax
from jax.experimental import pallas as pl
from jax.experimental.pallas import tpu as pltpu
from jax.experimental.pallas import tpu_sc as plsc

B = 32
N = 3125
NP = 3200
PRE_K = 2000
POST_K = 300
ROWS_PAD = 304
ROW_W = 8
THRESH = 0.7
SZM1 = 254.0
NEG = -1.0e30
INT_MIN = -2147483648

SORT_N = 2048
SORT_V = SORT_N // 16
SORT_PAD = SORT_N + 16
KEPT_PAD = 320


def _f32_sort_key(s):
    u = lax.bitcast_convert_type(s, jnp.int32)
    flipped = jnp.bitwise_xor(jnp.invert(u), INT_MIN)
    return jnp.where(u >= 0, u, flipped)


def _decode_body(cls0_ref, cls1_ref, d0_ref, d1_ref, d2_ref, d3_ref, anc_ref,
                 os_ref, ok_ref, ox1_ref, oy1_ref, ox2_ref, oy2_ref, othr_ref):
    c0 = cls0_ref[:, :]
    c1 = cls1_ref[:, :]
    m = jnp.maximum(c0, c1)
    s = (c1 - m) - jnp.log(jnp.exp(c0 - m) + jnp.exp(c1 - m))
    col = lax.broadcasted_iota(jnp.int32, (B, NP), 1)
    s = jnp.where(col < N, s, NEG)

    aw = anc_ref[2:3, :] - anc_ref[0:1, :] + 1.0
    ah = anc_ref[3:4, :] - anc_ref[1:2, :] + 1.0
    acx = anc_ref[0:1, :] + 0.5 * aw
    acy = anc_ref[1:2, :] + 0.5 * ah
    pcx = d0_ref[:, :] * aw + acx
    pcy = d1_ref[:, :] * ah + acy
    pw = jnp.exp(d2_ref[:, :]) * aw
    ph = jnp.exp(d3_ref[:, :]) * ah
    x1 = jnp.clip(pcx - 0.5 * pw, 0.0, SZM1)
    y1 = jnp.clip(pcy - 0.5 * ph, 0.0, SZM1)
    x2 = jnp.clip(pcx + 0.5 * pw, 0.0, SZM1)
    y2 = jnp.clip(pcy + 0.5 * ph, 0.0, SZM1)

    key = _f32_sort_key(s)
    lo0 = jnp.min(key, axis=1, keepdims=True)
    hi0 = jnp.max(key, axis=1, keepdims=True) + 1

    def bs_step(_, lh):
        lo, hi = lh
        mid = lo + ((hi - lo) >> 1)
        cnt = jnp.sum((key >= mid).astype(jnp.int32), axis=1, keepdims=True)
        ge = cnt >= PRE_K
        return jnp.where(ge, mid, lo), jnp.where(ge, hi, mid)

    lo, _ = lax.fori_loop(0, 32, bs_step, (lo0, hi0))

    os_ref[:, :] = s
    ok_ref[:, :] = key
    ox1_ref[:, :] = x1
    oy1_ref[:, :] = y1
    ox2_ref[:, :] = x2
    oy2_ref[:, :] = y2
    othr_ref[:, :] = jnp.broadcast_to(lo, (B, 128))


def _sc_body(s_hbm, k_hbm, x1_hbm, y1_hbm, x2_hbm, y2_hbm, thr_hbm,
             out_hbm,
             s_sc, k_sc, x1_sc, y1_sc, x2_sc, y2_sc, thr_sc,
             skeys, sidx, kx1, ky1, kx2, ky2, ka, rows, sem):
    wid = lax.axis_index("s") * 2 + lax.axis_index("c")

    cps = [
        pltpu.async_copy(s_hbm.at[wid], s_sc, sem),
        pltpu.async_copy(k_hbm.at[wid], k_sc, sem),
        pltpu.async_copy(x1_hbm.at[wid], x1_sc, sem),
        pltpu.async_copy(y1_hbm.at[wid], y1_sc, sem),
        pltpu.async_copy(x2_hbm.at[wid], x2_sc, sem),
        pltpu.async_copy(y2_hbm.at[wid], y2_sc, sem),
        pltpu.async_copy(thr_hbm.at[wid], thr_sc, sem),
    ]

    iota16 = lax.iota(jnp.int32, 16)
    zero16i = jnp.zeros((16,), jnp.int32)
    lane0 = iota16 == 0
    lane8 = iota16 < 8

    def pre_sort(v, _):
        skeys[pl.ds(v * 16, 16)] = jnp.full((16,), INT_MIN, jnp.int32)
        sidx[pl.ds(v * 16, 16)] = zero16i
        return 0

    lax.fori_loop(0, SORT_PAD // 16, pre_sort, 0)

    def pre_kept(v, _):
        off = v * 16
        kx1[pl.ds(off, 16)] = jnp.full((16,), 30000.0, jnp.float32)
        ky1[pl.ds(off, 16)] = jnp.full((16,), 30000.0, jnp.float32)
        kx2[pl.ds(off, 16)] = jnp.full((16,), 20000.0, jnp.float32)
        ky2[pl.ds(off, 16)] = jnp.full((16,), 20000.0, jnp.float32)
        ka[pl.ds(off, 16)] = jnp.full((16,), 1.0, jnp.float32)
        return 0

    lax.fori_loop(0, KEPT_PAD // 16, pre_kept, 0)

    def pre_rows(v, _):
        rows[pl.ds(v * 16, 16)] = jnp.zeros((16,), jnp.float32)
        return 0

    lax.fori_loop(0, (ROWS_PAD * ROW_W) // 16, pre_rows, 0)

    for cp in cps:
        cp.wait()

    thr = thr_sc[pl.ds(0, 16)]

    def comp(g, cnt):
        k = k_sc[pl.ds(g * 16, 16)]
        msk = k >= thr
        idx = iota16 + jnp.full((16,), g * 16, jnp.int32)
        mi = msk.astype(jnp.int32)
        pos = jnp.full((16,), cnt, jnp.int32) + plsc.cumsum(mi) - 1
        pos = jnp.minimum(pos, SORT_PAD - 1)
        plsc.store_scatter(skeys, [pos], k, mask=msk)
        plsc.store_scatter(sidx, [pos], idx, mask=msk)
        return cnt + jnp.sum(mi)

    cnt = lax.fori_loop(0, NP // 16, comp, jnp.int32(0))
    cnt = jnp.minimum(cnt, SORT_N)

    def vsort_all(v, _):
        off = v * 16
        k = skeys[pl.ds(off, 16)]
        p = sidx[pl.ds(off, 16)]
        ks, ps = plsc.sort_key_val(k, p, descending=True)
        skeys[pl.ds(off, 16)] = ks
        sidx[pl.ds(off, 16)] = ps
        return 0

    lax.fori_loop(0, SORT_V, vsort_all, 0)

    r = 1
    while r < SORT_V:
        nm = SORT_V // (2 * r)

        def revb(mi, _, r=r):
            base = mi * 2 * r + r
            if r == 1:
                off = base * 16
                skeys[pl.ds(off, 16)] = jnp.flip(skeys[pl.ds(off, 16)])
                sidx[pl.ds(off, 16)] = jnp.flip(sidx[pl.ds(off, 16)])
            else:
                for j in range(r // 2):
                    o1 = (base + j) * 16
                    o2 = (base + r - 1 - j) * 16
                    ka_ = skeys[pl.ds(o1, 16)]
                    kb_ = skeys[pl.ds(o2, 16)]
                    pa_ = sidx[pl.ds(o1, 16)]
                    pb_ = sidx[pl.ds(o2, 16)]
                    skeys[pl.ds(o1, 16)] = jnp.flip(kb_)
                    skeys[pl.ds(o2, 16)] = jnp.flip(ka_)
                    sidx[pl.ds(o1, 16)] = jnp.flip(pb_)
                    sidx[pl.ds(o2, 16)] = jnp.flip(pa_)
            return 0

        lax.fori_loop(0, nm, revb, 0)

        d = r
        while d >= 1:
            def ce(p, _, d=d):
                i = (p // d) * (2 * d) + (p % d)
                oa = i * 16
                ob = (i + d) * 16
                ka_ = skeys[pl.ds(oa, 16)]
                kb_ = skeys[pl.ds(ob, 16)]
                pa_ = sidx[pl.ds(oa, 16)]
                pb_ = sidx[pl.ds(ob, 16)]
                m_ = ka_ >= kb_
                skeys[pl.ds(oa, 16)] = jnp.where(m_, ka_, kb_)
                skeys[pl.ds(ob, 16)] = jnp.where(m_, kb_, ka_)
                sidx[pl.ds(oa, 16)] = jnp.where(m_, pa_, pb_)
                sidx[pl.ds(ob, 16)] = jnp.where(m_, pb_, pa_)
                return 0

            lax.fori_loop(0, SORT_V // 2, ce, 0)
            d //= 2

        lax.fori_loop(0, SORT_V, vsort_all, 0)
        r *= 2

    for parity in (0, 1, 0, 1):
        def tie_fix(t, _, parity=parity):
            ev = jnp.full((16,), t * 32 + parity, jnp.int32) + 2 * iota16
            od = ev + 1
            ka_ = plsc.load_gather(skeys, [ev])
            kb_ = plsc.load_gather(skeys, [od])
            pa_ = plsc.load_gather(sidx, [ev])
            pb_ = plsc.load_gather(sidx, [od])
            swap = (ka_ == kb_) & (pa_ > pb_)
            plsc.store_scatter(sidx, [ev], jnp.where(swap, pb_, pa_))
            plsc.store_scatter(sidx, [od], jnp.where(swap, pa_, pb_))
            return 0

        lax.fori_loop(0, SORT_N // 32, tie_fix, 0)

    jcap = jnp.minimum(cnt, PRE_K)

    def nms_cond(st):
        j, nk = st
        return jnp.logical_and(j < jcap, nk < POST_K)

    def nms_body(st):
        j, nk = st
        jv = jnp.full((16,), j, jnp.int32)
        gi = plsc.load_gather(sidx, [jv])
        cx1 = plsc.load_gather(x1_sc, [gi])
        cy1 = plsc.load_gather(y1_sc, [gi])
        cx2 = plsc.load_gather(x2_sc, [gi])
        cy2 = plsc.load_gather(y2_sc, [gi])
        cs = plsc.load_gather(s_sc, [gi])
        carea = (cx2 - cx1 + 1.0) * (cy2 - cy1 + 1.0)

        nch = (nk + 15) >> 4

        def inner(c, supp):
            off = c * 16
            kx1v = kx1[pl.ds(off, 16)]
            ky1v = ky1[pl.ds(off, 16)]
            kx2v = kx2[pl.ds(off, 16)]
            ky2v = ky2[pl.ds(off, 16)]
            kav = ka[pl.ds(off, 16)]
            xx1 = jnp.maximum(kx1v, cx1)
            yy1 = jnp.maximum(ky1v, cy1)
            xx2 = jnp.minimum(kx2v, cx2)
            yy2 = jnp.minimum(ky2v, cy2)
            iw = jnp.maximum(xx2 - xx1 + 1.0, 0.0)
            ih = jnp.maximum(yy2 - yy1 + 1.0, 0.0)
            inter = iw * ih
            iou = inter / ((carea + kav) - inter)
            bad = iou > THRESH
            return supp | bad.astype(jnp.int32)

        supp = lax.fori_loop(0, nch, inner, zero16i)
        keep_i = 1 - jnp.minimum(jnp.max(supp), 1)
        keepv = jnp.full((16,), keep_i, jnp.int32) > 0

        nkv = jnp.full((16,), nk, jnp.int32)
        m_app = lane0 & keepv
        plsc.store_scatter(kx1, [nkv], cx1, mask=m_app)
        plsc.store_scatter(ky1, [nkv], cy1, mask=m_app)
        plsc.store_scatter(kx2, [nkv], cx2, mask=m_app)
        plsc.store_scatter(ky2, [nkv], cy2, mask=m_app)
        plsc.store_scatter(ka, [nkv], carea, mask=m_app)

        rowv = jnp.where(iota16 == 0, cs,
               jnp.where(iota16 == 1, cx1,
               jnp.where(iota16 == 2, cy1,
               jnp.where(iota16 == 3, cx2,
               jnp.where(iota16 == 4, cy2,
                         jnp.zeros((16,), jnp.float32))))))
        ridx = nkv * ROW_W + iota16
        plsc.store_scatter(rows, [ridx], rowv, mask=lane8 & keepv)
        return j + 1, nk + keep_i

    lax.while_loop(nms_cond, nms_body, (jnp.int32(0), jnp.int32(0)))

    pltpu.sync_copy(rows, out_hbm.at[wid])


@jax.jit
def kernel(pred_cls, pred_reg, anchors):
    pc = pred_cls.reshape(B, 2, 5, 625)
    pr = pred_reg.reshape(B, 4, 5, 625)
    pad = ((0, 0), (0, NP - N))
    cls0 = jnp.pad(pc[:, 0].reshape(B, N), pad)
    cls1 = jnp.pad(pc[:, 1].reshape(B, N), pad)
    d0 = jnp.pad(pr[:, 0].reshape(B, N), pad)
    d1 = jnp.pad(pr[:, 1].reshape(B, N), pad)
    d2 = jnp.pad(pr[:, 2].reshape(B, N), pad)
    d3 = jnp.pad(pr[:, 3].reshape(B, N), pad)
    anc = jnp.pad(anchors.T, ((0, 0), (0, NP - N)))

    f32 = jnp.float32
    i32 = jnp.int32
    dec_sh = [
        jax.ShapeDtypeStruct((B, NP), f32),
        jax.ShapeDtypeStruct((B, NP), i32),
        jax.ShapeDtypeStruct((B, NP), f32),
        jax.ShapeDtypeStruct((B, NP), f32),
        jax.ShapeDtypeStruct((B, NP), f32),
        jax.ShapeDtypeStruct((B, NP), f32),
        jax.ShapeDtypeStruct((B, 128), i32),
    ]
    s, key, x1, y1, x2, y2, thr = pl.pallas_call(
        _decode_body, out_shape=dec_sh)(cls0, cls1, d0, d1, d2, d3, anc)

    mesh = plsc.VectorSubcoreMesh(core_axis_name="c", subcore_axis_name="s")
    sc = functools.partial(
        pl.kernel,
        mesh=mesh,
        compiler_params=pltpu.CompilerParams(needs_layout_passes=False),
        out_type=jax.ShapeDtypeStruct((B, ROWS_PAD * ROW_W), f32),
        scratch_types=[
            pltpu.VMEM((NP,), f32),
            pltpu.VMEM((NP,), i32),
            pltpu.VMEM((NP,), f32),
            pltpu.VMEM((NP,), f32),
            pltpu.VMEM((NP,), f32),
            pltpu.VMEM((NP,), f32),
            pltpu.VMEM((128,), i32),
            pltpu.VMEM((SORT_PAD,), i32),
            pltpu.VMEM((SORT_PAD,), i32),
            pltpu.VMEM((KEPT_PAD,), f32),
            pltpu.VMEM((KEPT_PAD,), f32),
            pltpu.VMEM((KEPT_PAD,), f32),
            pltpu.VMEM((KEPT_PAD,), f32),
            pltpu.VMEM((KEPT_PAD,), f32),
            pltpu.VMEM((ROWS_PAD * ROW_W,), f32),
            pltpu.SemaphoreType.DMA,
        ],
    )(_sc_body)
    rows = sc(s, key, x1, y1, x2, y2, thr)
    return rows.reshape(B, ROWS_PAD, ROW_W)[:, :POST_K, :5]


# speedup vs baseline: 46.9676x; 1.1308x over previous
import functools

import jax
import jax.numpy as jnp
from jax import lax
from jax.experimental import pallas as pl
from jax.experimental.pallas import tpu as pltpu
from jax.experimental.pallas import tpu_sc as plsc

B = 32
N = 3125
NP = 3200
PRE_K = 2000
POST_K = 300
ROWS_PAD = 304
ROW_W = 8
THRESH = 0.7
SZM1 = 254.0
NEG = -1.0e30
INT_MIN = -2147483648

SORT_N = 2048
SORT_V = SORT_N // 16
SORT_PAD = SORT_N + 16
KEPT_PAD = 320


def _f32_sort_key(s):
    u = lax.bitcast_convert_type(s, jnp.int32)
    flipped = jnp.bitwise_xor(jnp.invert(u), INT_MIN)
    return jnp.where(u >= 0, u, flipped)


def _decode_body(pc_ref, pr_ref, anc_ref,
                 os_ref, ok_ref, ox1_ref, oy1_ref, ox2_ref, oy2_ref, othr_ref):
    s_parts = []
    x1_parts = []
    y1_parts = []
    x2_parts = []
    y2_parts = []
    for a in range(5):
        c0 = pc_ref[:, a, :, :].reshape(B, 625)
        c1 = pc_ref[:, 5 + a, :, :].reshape(B, 625)
        m = jnp.maximum(c0, c1)
        s = (c1 - m) - jnp.log(jnp.exp(c0 - m) + jnp.exp(c1 - m))
        s_parts.append(s)

        ax1 = anc_ref[0:1, a, :]
        ay1 = anc_ref[1:2, a, :]
        ax2 = anc_ref[2:3, a, :]
        ay2 = anc_ref[3:4, a, :]
        aw = ax2 - ax1 + 1.0
        ah = ay2 - ay1 + 1.0
        acx = ax1 + 0.5 * aw
        acy = ay1 + 0.5 * ah
        d0 = pr_ref[:, a, :, :].reshape(B, 625)
        d1 = pr_ref[:, 5 + a, :, :].reshape(B, 625)
        d2 = pr_ref[:, 10 + a, :, :].reshape(B, 625)
        d3 = pr_ref[:, 15 + a, :, :].reshape(B, 625)
        pcx = d0 * aw + acx
        pcy = d1 * ah + acy
        pw = jnp.exp(d2) * aw
        ph = jnp.exp(d3) * ah
        x1_parts.append(jnp.clip(pcx - 0.5 * pw, 0.0, SZM1))
        y1_parts.append(jnp.clip(pcy - 0.5 * ph, 0.0, SZM1))
        x2_parts.append(jnp.clip(pcx + 0.5 * pw, 0.0, SZM1))
        y2_parts.append(jnp.clip(pcy + 0.5 * ph, 0.0, SZM1))

    padf = jnp.full((B, NP - N), NEG, jnp.float32)
    s = jnp.concatenate(s_parts + [padf], axis=1)
    padz = jnp.zeros((B, NP - N), jnp.float32)
    x1 = jnp.concatenate(x1_parts + [padz], axis=1)
    y1 = jnp.concatenate(y1_parts + [padz], axis=1)
    x2 = jnp.concatenate(x2_parts + [padz], axis=1)
    y2 = jnp.concatenate(y2_parts + [padz], axis=1)

    key = _f32_sort_key(s)
    lo0 = jnp.min(key, axis=1, keepdims=True)
    hi0 = jnp.max(key, axis=1, keepdims=True) + 1

    def bs_step(_, lh):
        lo, hi = lh
        mid = lo + ((hi - lo) >> 1)
        cnt = jnp.sum((key >= mid).astype(jnp.int32), axis=1, keepdims=True)
        ge = cnt >= PRE_K
        return jnp.where(ge, mid, lo), jnp.where(ge, hi, mid)

    lo, _ = lax.fori_loop(0, 32, bs_step, (lo0, hi0))

    os_ref[:, :] = s
    ok_ref[:, :] = key
    ox1_ref[:, :] = x1
    oy1_ref[:, :] = y1
    ox2_ref[:, :] = x2
    oy2_ref[:, :] = y2
    othr_ref[:, :] = jnp.broadcast_to(lo, (B, 128))


def _sc_body(s_hbm, k_hbm, x1_hbm, y1_hbm, x2_hbm, y2_hbm, thr_hbm,
             out_hbm,
             s_sc, k_sc, x1_sc, y1_sc, x2_sc, y2_sc, thr_sc,
             skeys, sidx, kx1, ky1, kx2, ky2, ka, rows, sem):
    wid = lax.axis_index("s") * 2 + lax.axis_index("c")

    cps = [
        pltpu.async_copy(s_hbm.at[wid], s_sc, sem),
        pltpu.async_copy(k_hbm.at[wid], k_sc, sem),
        pltpu.async_copy(x1_hbm.at[wid], x1_sc, sem),
        pltpu.async_copy(y1_hbm.at[wid], y1_sc, sem),
        pltpu.async_copy(x2_hbm.at[wid], x2_sc, sem),
        pltpu.async_copy(y2_hbm.at[wid], y2_sc, sem),
        pltpu.async_copy(thr_hbm.at[wid], thr_sc, sem),
    ]

    iota16 = lax.iota(jnp.int32, 16)
    zero16i = jnp.zeros((16,), jnp.int32)
    lane0 = iota16 == 0
    lane8 = iota16 < 8

    def pre_sort(v, _):
        skeys[pl.ds(v * 16, 16)] = jnp.full((16,), INT_MIN, jnp.int32)
        sidx[pl.ds(v * 16, 16)] = zero16i
        return 0

    lax.fori_loop(0, SORT_PAD // 16, pre_sort, 0)

    def pre_kept(v, _):
        off = v * 16
        kx1[pl.ds(off, 16)] = jnp.full((16,), 30000.0, jnp.float32)
        ky1[pl.ds(off, 16)] = jnp.full((16,), 30000.0, jnp.float32)
        kx2[pl.ds(off, 16)] = jnp.full((16,), 20000.0, jnp.float32)
        ky2[pl.ds(off, 16)] = jnp.full((16,), 20000.0, jnp.float32)
        ka[pl.ds(off, 16)] = jnp.full((16,), 1.0, jnp.float32)
        return 0

    lax.fori_loop(0, KEPT_PAD // 16, pre_kept, 0)

    def pre_rows(v, _):
        rows[pl.ds(v * 16, 16)] = jnp.zeros((16,), jnp.float32)
        return 0

    lax.fori_loop(0, (ROWS_PAD * ROW_W) // 16, pre_rows, 0)

    for cp in cps:
        cp.wait()

    thr = thr_sc[pl.ds(0, 16)]

    def comp(g, cnt):
        k = k_sc[pl.ds(g * 16, 16)]
        msk = k >= thr
        idx = iota16 + jnp.full((16,), g * 16, jnp.int32)
        mi = msk.astype(jnp.int32)
        pos = jnp.full((16,), cnt, jnp.int32) + plsc.cumsum(mi) - 1
        pos = jnp.minimum(pos, SORT_PAD - 1)
        plsc.store_scatter(skeys, [pos], k, mask=msk)
        plsc.store_scatter(sidx, [pos], idx, mask=msk)
        return cnt + jnp.sum(mi)

    cnt = lax.fori_loop(0, NP // 16, comp, jnp.int32(0))
    cnt = jnp.minimum(cnt, SORT_N)

    def vsort_all(v, _):
        off = v * 16
        k = skeys[pl.ds(off, 16)]
        p = sidx[pl.ds(off, 16)]
        ks, ps = plsc.sort_key_val(k, p, descending=True)
        skeys[pl.ds(off, 16)] = ks
        sidx[pl.ds(off, 16)] = ps
        return 0

    lax.fori_loop(0, SORT_V, vsort_all, 0)

    r = 1
    while r < SORT_V:
        nm = SORT_V // (2 * r)

        def revb(mi, _, r=r):
            base = mi * 2 * r + r
            if r == 1:
                off = base * 16
                skeys[pl.ds(off, 16)] = jnp.flip(skeys[pl.ds(off, 16)])
                sidx[pl.ds(off, 16)] = jnp.flip(sidx[pl.ds(off, 16)])
            else:
                for j in range(r // 2):
                    o1 = (base + j) * 16
                    o2 = (base + r - 1 - j) * 16
                    ka_ = skeys[pl.ds(o1, 16)]
                    kb_ = skeys[pl.ds(o2, 16)]
                    pa_ = sidx[pl.ds(o1, 16)]
                    pb_ = sidx[pl.ds(o2, 16)]
                    skeys[pl.ds(o1, 16)] = jnp.flip(kb_)
                    skeys[pl.ds(o2, 16)] = jnp.flip(ka_)
                    sidx[pl.ds(o1, 16)] = jnp.flip(pb_)
                    sidx[pl.ds(o2, 16)] = jnp.flip(pa_)
            return 0

        lax.fori_loop(0, nm, revb, 0)

        d = r
        while d >= 1:
            def ce(p, _, d=d):
                i = (p // d) * (2 * d) + (p % d)
                oa = i * 16
                ob = (i + d) * 16
                ka_ = skeys[pl.ds(oa, 16)]
                kb_ = skeys[pl.ds(ob, 16)]
                pa_ = sidx[pl.ds(oa, 16)]
                pb_ = sidx[pl.ds(ob, 16)]
                m_ = ka_ >= kb_
                skeys[pl.ds(oa, 16)] = jnp.where(m_, ka_, kb_)
                skeys[pl.ds(ob, 16)] = jnp.where(m_, kb_, ka_)
                sidx[pl.ds(oa, 16)] = jnp.where(m_, pa_, pb_)
                sidx[pl.ds(ob, 16)] = jnp.where(m_, pb_, pa_)
                return 0

            lax.fori_loop(0, SORT_V // 2, ce, 0)
            d //= 2

        lax.fori_loop(0, SORT_V, vsort_all, 0)
        r *= 2

    for parity in (0, 1, 0, 1):
        def tie_fix(t, _, parity=parity):
            ev = jnp.full((16,), t * 32 + parity, jnp.int32) + 2 * iota16
            od = ev + 1
            ka_ = plsc.load_gather(skeys, [ev])
            kb_ = plsc.load_gather(skeys, [od])
            pa_ = plsc.load_gather(sidx, [ev])
            pb_ = plsc.load_gather(sidx, [od])
            swap = (ka_ == kb_) & (pa_ > pb_)
            plsc.store_scatter(sidx, [ev], jnp.where(swap, pb_, pa_))
            plsc.store_scatter(sidx, [od], jnp.where(swap, pa_, pb_))
            return 0

        lax.fori_loop(0, SORT_N // 32, tie_fix, 0)

    jcap = jnp.minimum(cnt, PRE_K)

    def nms_cond(st):
        j, nk = st
        return jnp.logical_and(j < jcap, nk < POST_K)

    def nms_body(st):
        j, nk = st
        jv = jnp.full((16,), j, jnp.int32)
        gi = plsc.load_gather(sidx, [jv])
        cx1 = plsc.load_gather(x1_sc, [gi])
        cy1 = plsc.load_gather(y1_sc, [gi])
        cx2 = plsc.load_gather(x2_sc, [gi])
        cy2 = plsc.load_gather(y2_sc, [gi])
        cs = plsc.load_gather(s_sc, [gi])
        carea = (cx2 - cx1 + 1.0) * (cy2 - cy1 + 1.0)

        nch = (nk + 15) >> 4

        def inner(c, supp):
            off = c * 16
            kx1v = kx1[pl.ds(off, 16)]
            ky1v = ky1[pl.ds(off, 16)]
            kx2v = kx2[pl.ds(off, 16)]
            ky2v = ky2[pl.ds(off, 16)]
            kav = ka[pl.ds(off, 16)]
            xx1 = jnp.maximum(kx1v, cx1)
            yy1 = jnp.maximum(ky1v, cy1)
            xx2 = jnp.minimum(kx2v, cx2)
            yy2 = jnp.minimum(ky2v, cy2)
            iw = jnp.maximum(xx2 - xx1 + 1.0, 0.0)
            ih = jnp.maximum(yy2 - yy1 + 1.0, 0.0)
            inter = iw * ih
            iou = inter / ((carea + kav) - inter)
            bad = iou > THRESH
            return supp | bad.astype(jnp.int32)

        supp = lax.fori_loop(0, nch, inner, zero16i)
        keep_i = 1 - jnp.minimum(jnp.max(supp), 1)
        keepv = jnp.full((16,), keep_i, jnp.int32) > 0

        nkv = jnp.full((16,), nk, jnp.int32)
        m_app = lane0 & keepv
        plsc.store_scatter(kx1, [nkv], cx1, mask=m_app)
        plsc.store_scatter(ky1, [nkv], cy1, mask=m_app)
        plsc.store_scatter(kx2, [nkv], cx2, mask=m_app)
        plsc.store_scatter(ky2, [nkv], cy2, mask=m_app)
        plsc.store_scatter(ka, [nkv], carea, mask=m_app)

        rowv = jnp.where(iota16 == 0, cs,
               jnp.where(iota16 == 1, cx1,
               jnp.where(iota16 == 2, cy1,
               jnp.where(iota16 == 3, cx2,
               jnp.where(iota16 == 4, cy2,
                         jnp.zeros((16,), jnp.float32))))))
        ridx = nkv * ROW_W + iota16
        plsc.store_scatter(rows, [ridx], rowv, mask=lane8 & keepv)
        return j + 1, nk + keep_i

    lax.while_loop(nms_cond, nms_body, (jnp.int32(0), jnp.int32(0)))

    pltpu.sync_copy(rows, out_hbm.at[wid])


@jax.jit
def kernel(pred_cls, pred_reg, anchors):
    anc = anchors.T.reshape(4, 5, 625)

    f32 = jnp.float32
    i32 = jnp.int32
    dec_sh = [
        jax.ShapeDtypeStruct((B, NP), f32),
        jax.ShapeDtypeStruct((B, NP), i32),
        jax.ShapeDtypeStruct((B, NP), f32),
        jax.ShapeDtypeStruct((B, NP), f32),
        jax.ShapeDtypeStruct((B, NP), f32),
        jax.ShapeDtypeStruct((B, NP), f32),
        jax.ShapeDtypeStruct((B, 128), i32),
    ]
    s, key, x1, y1, x2, y2, thr = pl.pallas_call(
        _decode_body, out_shape=dec_sh)(pred_cls, pred_reg, anc)

    mesh = plsc.VectorSubcoreMesh(core_axis_name="c", subcore_axis_name="s")
    sc = functools.partial(
        pl.kernel,
        mesh=mesh,
        compiler_params=pltpu.CompilerParams(needs_layout_passes=False),
        out_type=jax.ShapeDtypeStruct((B, ROWS_PAD * ROW_W), f32),
        scratch_types=[
            pltpu.VMEM((NP,), f32),
            pltpu.VMEM((NP,), i32),
            pltpu.VMEM((NP,), f32),
            pltpu.VMEM((NP,), f32),
            pltpu.VMEM((NP,), f32),
            pltpu.VMEM((NP,), f32),
            pltpu.VMEM((128,), i32),
            pltpu.VMEM((SORT_PAD,), i32),
            pltpu.VMEM((SORT_PAD,), i32),
            pltpu.VMEM((KEPT_PAD,), f32),
            pltpu.VMEM((KEPT_PAD,), f32),
            pltpu.VMEM((KEPT_PAD,), f32),
            pltpu.VMEM((KEPT_PAD,), f32),
            pltpu.VMEM((KEPT_PAD,), f32),
            pltpu.VMEM((ROWS_PAD * ROW_W,), f32),
            pltpu.SemaphoreType.DMA,
        ],
    )(_sc_body)
    rows = sc(s, key, x1, y1, x2, y2, thr)
    return rows.reshape(B, ROWS_PAD, ROW_W)[:, :POST_K, :5]


# speedup vs baseline: 52.3307x; 1.1142x over previous
import functools

import jax
import jax.numpy as jnp
from jax import lax
from jax.experimental import pallas as pl
from jax.experimental.pallas import tpu as pltpu
from jax.experimental.pallas import tpu_sc as plsc

B = 32
N = 3125
NP = 3200
PRE_K = 2000
POST_K = 300
ROWS_PAD = 304
ROW_W = 8
THRESH = 0.7
SZM1 = 254.0
NEG = -1.0e30
INT_MIN = -2147483648

SORT_N = 2048
SORT_V = SORT_N // 16
SORT_PAD = SORT_N + 16
KEPT_PAD = 320


def _f32_sort_key(s):
    u = lax.bitcast_convert_type(s, jnp.int32)
    flipped = jnp.bitwise_xor(jnp.invert(u), INT_MIN)
    return jnp.where(u >= 0, u, flipped)


def _decode_body(pc_ref, pr_ref, anc_ref,
                 os_ref, ok_ref, ox1_ref, oy1_ref, ox2_ref, oy2_ref, othr_ref):
    s_parts = []
    x1_parts = []
    y1_parts = []
    x2_parts = []
    y2_parts = []
    for a in range(5):
        c0 = pc_ref[:, a, :]
        c1 = pc_ref[:, 5 + a, :]
        m = jnp.maximum(c0, c1)
        s = (c1 - m) - jnp.log(jnp.exp(c0 - m) + jnp.exp(c1 - m))
        s_parts.append(s)

        ax1 = anc_ref[0:1, a, :]
        ay1 = anc_ref[1:2, a, :]
        ax2 = anc_ref[2:3, a, :]
        ay2 = anc_ref[3:4, a, :]
        aw = ax2 - ax1 + 1.0
        ah = ay2 - ay1 + 1.0
        acx = ax1 + 0.5 * aw
        acy = ay1 + 0.5 * ah
        d0 = pr_ref[:, a, :]
        d1 = pr_ref[:, 5 + a, :]
        d2 = pr_ref[:, 10 + a, :]
        d3 = pr_ref[:, 15 + a, :]
        pcx = d0 * aw + acx
        pcy = d1 * ah + acy
        pw = jnp.exp(d2) * aw
        ph = jnp.exp(d3) * ah
        x1_parts.append(jnp.clip(pcx - 0.5 * pw, 0.0, SZM1))
        y1_parts.append(jnp.clip(pcy - 0.5 * ph, 0.0, SZM1))
        x2_parts.append(jnp.clip(pcx + 0.5 * pw, 0.0, SZM1))
        y2_parts.append(jnp.clip(pcy + 0.5 * ph, 0.0, SZM1))

    padf = jnp.full((B, NP - N), NEG, jnp.float32)
    s = jnp.concatenate(s_parts + [padf], axis=1)
    padz = jnp.zeros((B, NP - N), jnp.float32)
    x1 = jnp.concatenate(x1_parts + [padz], axis=1)
    y1 = jnp.concatenate(y1_parts + [padz], axis=1)
    x2 = jnp.concatenate(x2_parts + [padz], axis=1)
    y2 = jnp.concatenate(y2_parts + [padz], axis=1)

    key = _f32_sort_key(s)
    lo0 = jnp.min(key, axis=1, keepdims=True)
    hi0 = jnp.max(key, axis=1, keepdims=True) + 1

    def bs_step(_, lh):
        lo, hi = lh
        mid = lo + ((hi - lo) >> 1)
        cnt = jnp.sum((key >= mid).astype(jnp.int32), axis=1, keepdims=True)
        ge = cnt >= PRE_K
        return jnp.where(ge, mid, lo), jnp.where(ge, hi, mid)

    lo, _ = lax.fori_loop(0, 32, bs_step, (lo0, hi0))

    os_ref[:, :] = s
    ok_ref[:, :] = key
    ox1_ref[:, :] = x1
    oy1_ref[:, :] = y1
    ox2_ref[:, :] = x2
    oy2_ref[:, :] = y2
    othr_ref[:, :] = jnp.broadcast_to(lo, (B, 128))


def _sc_body(s_hbm, k_hbm, x1_hbm, y1_hbm, x2_hbm, y2_hbm, thr_hbm,
             out_hbm,
             s_sc, k_sc, x1_sc, y1_sc, x2_sc, y2_sc, thr_sc,
             skeys, sidx, kx1, ky1, kx2, ky2, ka, rows, sem):
    wid = lax.axis_index("s") * 2 + lax.axis_index("c")

    cps = [
        pltpu.async_copy(s_hbm.at[wid], s_sc, sem),
        pltpu.async_copy(k_hbm.at[wid], k_sc, sem),
        pltpu.async_copy(x1_hbm.at[wid], x1_sc, sem),
        pltpu.async_copy(y1_hbm.at[wid], y1_sc, sem),
        pltpu.async_copy(x2_hbm.at[wid], x2_sc, sem),
        pltpu.async_copy(y2_hbm.at[wid], y2_sc, sem),
        pltpu.async_copy(thr_hbm.at[wid], thr_sc, sem),
    ]

    iota16 = lax.iota(jnp.int32, 16)
    zero16i = jnp.zeros((16,), jnp.int32)
    lane0 = iota16 == 0
    lane8 = iota16 < 8

    def pre_sort(v, _):
        skeys[pl.ds(v * 16, 16)] = jnp.full((16,), INT_MIN, jnp.int32)
        sidx[pl.ds(v * 16, 16)] = zero16i
        return 0

    lax.fori_loop(0, SORT_PAD // 16, pre_sort, 0)

    def pre_kept(v, _):
        off = v * 16
        kx1[pl.ds(off, 16)] = jnp.full((16,), 30000.0, jnp.float32)
        ky1[pl.ds(off, 16)] = jnp.full((16,), 30000.0, jnp.float32)
        kx2[pl.ds(off, 16)] = jnp.full((16,), 20000.0, jnp.float32)
        ky2[pl.ds(off, 16)] = jnp.full((16,), 20000.0, jnp.float32)
        ka[pl.ds(off, 16)] = jnp.full((16,), 1.0, jnp.float32)
        return 0

    lax.fori_loop(0, KEPT_PAD // 16, pre_kept, 0)

    def pre_rows(v, _):
        rows[pl.ds(v * 16, 16)] = jnp.zeros((16,), jnp.float32)
        return 0

    lax.fori_loop(0, (ROWS_PAD * ROW_W) // 16, pre_rows, 0)

    for cp in cps:
        cp.wait()

    thr = thr_sc[pl.ds(0, 16)]

    def comp(g, cnt):
        k = k_sc[pl.ds(g * 16, 16)]
        msk = k >= thr
        idx = iota16 + jnp.full((16,), g * 16, jnp.int32)
        mi = msk.astype(jnp.int32)
        pos = jnp.full((16,), cnt, jnp.int32) + plsc.cumsum(mi) - 1
        pos = jnp.minimum(pos, SORT_PAD - 1)
        plsc.store_scatter(skeys, [pos], k, mask=msk)
        plsc.store_scatter(sidx, [pos], idx, mask=msk)
        return cnt + jnp.sum(mi)

    cnt = lax.fori_loop(0, NP // 16, comp, jnp.int32(0))
    cnt = jnp.minimum(cnt, SORT_N)

    def vsort_all(v, _):
        off = v * 16
        k = skeys[pl.ds(off, 16)]
        p = sidx[pl.ds(off, 16)]
        ks, ps = plsc.sort_key_val(k, p, descending=True)
        skeys[pl.ds(off, 16)] = ks
        sidx[pl.ds(off, 16)] = ps
        return 0

    lax.fori_loop(0, SORT_V, vsort_all, 0)

    r = 1
    while r < SORT_V:
        nm = SORT_V // (2 * r)

        def revb(mi, _, r=r):
            base = mi * 2 * r + r
            if r == 1:
                off = base * 16
                skeys[pl.ds(off, 16)] = jnp.flip(skeys[pl.ds(off, 16)])
                sidx[pl.ds(off, 16)] = jnp.flip(sidx[pl.ds(off, 16)])
            else:
                for j in range(r // 2):
                    o1 = (base + j) * 16
                    o2 = (base + r - 1 - j) * 16
                    ka_ = skeys[pl.ds(o1, 16)]
                    kb_ = skeys[pl.ds(o2, 16)]
                    pa_ = sidx[pl.ds(o1, 16)]
                    pb_ = sidx[pl.ds(o2, 16)]
                    skeys[pl.ds(o1, 16)] = jnp.flip(kb_)
                    skeys[pl.ds(o2, 16)] = jnp.flip(ka_)
                    sidx[pl.ds(o1, 16)] = jnp.flip(pb_)
                    sidx[pl.ds(o2, 16)] = jnp.flip(pa_)
            return 0

        lax.fori_loop(0, nm, revb, 0)

        d = r
        while d >= 1:
            def ce(p, _, d=d):
                i = (p // d) * (2 * d) + (p % d)
                oa = i * 16
                ob = (i + d) * 16
                ka_ = skeys[pl.ds(oa, 16)]
                kb_ = skeys[pl.ds(ob, 16)]
                pa_ = sidx[pl.ds(oa, 16)]
                pb_ = sidx[pl.ds(ob, 16)]
                m_ = ka_ >= kb_
                skeys[pl.ds(oa, 16)] = jnp.where(m_, ka_, kb_)
                skeys[pl.ds(ob, 16)] = jnp.where(m_, kb_, ka_)
                sidx[pl.ds(oa, 16)] = jnp.where(m_, pa_, pb_)
                sidx[pl.ds(ob, 16)] = jnp.where(m_, pb_, pa_)
                return 0

            lax.fori_loop(0, SORT_V // 2, ce, 0)
            d //= 2

        lax.fori_loop(0, SORT_V, vsort_all, 0)
        r *= 2

    for parity in (0, 1, 0, 1):
        def tie_fix(t, _, parity=parity):
            ev = jnp.full((16,), t * 32 + parity, jnp.int32) + 2 * iota16
            od = ev + 1
            ka_ = plsc.load_gather(skeys, [ev])
            kb_ = plsc.load_gather(skeys, [od])
            pa_ = plsc.load_gather(sidx, [ev])
            pb_ = plsc.load_gather(sidx, [od])
            swap = (ka_ == kb_) & (pa_ > pb_)
            plsc.store_scatter(sidx, [ev], jnp.where(swap, pb_, pa_))
            plsc.store_scatter(sidx, [od], jnp.where(swap, pa_, pb_))
            return 0

        lax.fori_loop(0, SORT_N // 32, tie_fix, 0)

    jcap = jnp.minimum(cnt, PRE_K)

    def nms_cond(st):
        j, nk = st
        return jnp.logical_and(j < jcap, nk < POST_K)

    def nms_body(st):
        j, nk = st
        jv = jnp.full((16,), j, jnp.int32)
        gi = plsc.load_gather(sidx, [jv])
        cx1 = plsc.load_gather(x1_sc, [gi])
        cy1 = plsc.load_gather(y1_sc, [gi])
        cx2 = plsc.load_gather(x2_sc, [gi])
        cy2 = plsc.load_gather(y2_sc, [gi])
        cs = plsc.load_gather(s_sc, [gi])
        carea = (cx2 - cx1 + 1.0) * (cy2 - cy1 + 1.0)

        nch = (nk + 15) >> 4

        def inner(c, supp):
            off = c * 16
            kx1v = kx1[pl.ds(off, 16)]
            ky1v = ky1[pl.ds(off, 16)]
            kx2v = kx2[pl.ds(off, 16)]
            ky2v = ky2[pl.ds(off, 16)]
            kav = ka[pl.ds(off, 16)]
            xx1 = jnp.maximum(kx1v, cx1)
            yy1 = jnp.maximum(ky1v, cy1)
            xx2 = jnp.minimum(kx2v, cx2)
            yy2 = jnp.minimum(ky2v, cy2)
            iw = jnp.maximum(xx2 - xx1 + 1.0, 0.0)
            ih = jnp.maximum(yy2 - yy1 + 1.0, 0.0)
            inter = iw * ih
            iou = inter / ((carea + kav) - inter)
            bad = iou > THRESH
            return supp | bad.astype(jnp.int32)

        supp = lax.fori_loop(0, nch, inner, zero16i)
        keep_i = 1 - jnp.minimum(jnp.max(supp), 1)
        keepv = jnp.full((16,), keep_i, jnp.int32) > 0

        nkv = jnp.full((16,), nk, jnp.int32)
        m_app = lane0 & keepv
        plsc.store_scatter(kx1, [nkv], cx1, mask=m_app)
        plsc.store_scatter(ky1, [nkv], cy1, mask=m_app)
        plsc.store_scatter(kx2, [nkv], cx2, mask=m_app)
        plsc.store_scatter(ky2, [nkv], cy2, mask=m_app)
        plsc.store_scatter(ka, [nkv], carea, mask=m_app)

        rowv = jnp.where(iota16 == 0, cs,
               jnp.where(iota16 == 1, cx1,
               jnp.where(iota16 == 2, cy1,
               jnp.where(iota16 == 3, cx2,
               jnp.where(iota16 == 4, cy2,
                         jnp.zeros((16,), jnp.float32))))))
        ridx = nkv * ROW_W + iota16
        plsc.store_scatter(rows, [ridx], rowv, mask=lane8 & keepv)
        return j + 1, nk + keep_i

    lax.while_loop(nms_cond, nms_body, (jnp.int32(0), jnp.int32(0)))

    pltpu.sync_copy(rows, out_hbm.at[wid])


@jax.jit
def kernel(pred_cls, pred_reg, anchors):
    anc = anchors.T.reshape(4, 5, 625)

    f32 = jnp.float32
    i32 = jnp.int32
    dec_sh = [
        jax.ShapeDtypeStruct((B, NP), f32),
        jax.ShapeDtypeStruct((B, NP), i32),
        jax.ShapeDtypeStruct((B, NP), f32),
        jax.ShapeDtypeStruct((B, NP), f32),
        jax.ShapeDtypeStruct((B, NP), f32),
        jax.ShapeDtypeStruct((B, NP), f32),
        jax.ShapeDtypeStruct((B, 128), i32),
    ]
    s, key, x1, y1, x2, y2, thr = pl.pallas_call(
        _decode_body, out_shape=dec_sh)(
            pred_cls.reshape(B, 10, 625), pred_reg.reshape(B, 20, 625), anc)

    mesh = plsc.VectorSubcoreMesh(core_axis_name="c", subcore_axis_name="s")
    sc = functools.partial(
        pl.kernel,
        mesh=mesh,
        compiler_params=pltpu.CompilerParams(needs_layout_passes=False),
        out_type=jax.ShapeDtypeStruct((B, ROWS_PAD * ROW_W), f32),
        scratch_types=[
            pltpu.VMEM((NP,), f32),
            pltpu.VMEM((NP,), i32),
            pltpu.VMEM((NP,), f32),
            pltpu.VMEM((NP,), f32),
            pltpu.VMEM((NP,), f32),
            pltpu.VMEM((NP,), f32),
            pltpu.VMEM((128,), i32),
            pltpu.VMEM((SORT_PAD,), i32),
            pltpu.VMEM((SORT_PAD,), i32),
            pltpu.VMEM((KEPT_PAD,), f32),
            pltpu.VMEM((KEPT_PAD,), f32),
            pltpu.VMEM((KEPT_PAD,), f32),
            pltpu.VMEM((KEPT_PAD,), f32),
            pltpu.VMEM((KEPT_PAD,), f32),
            pltpu.VMEM((ROWS_PAD * ROW_W,), f32),
            pltpu.SemaphoreType.DMA,
        ],
    )(_sc_body)
    rows = sc(s, key, x1, y1, x2, y2, thr)
    return rows.reshape(B, ROWS_PAD, ROW_W)[:, :POST_K, :5]
